# Initial kernel scaffold; baseline (speedup 1.0000x reference)
import jax
import jax.numpy as jnp
from jax.experimental import pallas as pl


def kernel(x, embedding):
    raise NotImplementedError("write your pallas kernel here")


# speedup vs baseline: 4.8078x; 4.8078x over previous
import functools

import jax
import jax.numpy as jnp
from jax import lax
from jax.experimental import pallas as pl
from jax.experimental.pallas import tpu as pltpu
from jax.experimental.pallas import tpu_sc as plsc

_NUM_CORES = 2
_NUM_SUBCORES = 16
_NUM_WORKERS = _NUM_CORES * _NUM_SUBCORES

_CHUNK = 1024


@functools.cache
def _gather_call(n: int, v: int, d: int):
    assert n % (_NUM_WORKERS * _CHUNK) == 0
    per_worker = n // _NUM_WORKERS
    n_chunks = per_worker // _CHUNK
    mesh = plsc.VectorSubcoreMesh(core_axis_name="c", subcore_axis_name="s")

    def body(idx_hbm, table_hbm, out_hbm, idx_v, rows_v, sem):
        wid = lax.axis_index("s") * _NUM_CORES + lax.axis_index("c")
        base = wid * per_worker

        def step(i, carry):
            off = base + i * _CHUNK
            pltpu.sync_copy(idx_hbm.at[pl.ds(off, _CHUNK)], idx_v)
            pltpu.async_copy(table_hbm.at[idx_v], rows_v, sem).wait()
            pltpu.sync_copy(rows_v, out_hbm.at[pl.ds(off, _CHUNK)])
            return carry

        lax.fori_loop(0, n_chunks, step, 0)

    return pl.kernel(
        body,
        out_type=jax.ShapeDtypeStruct((n, d), jnp.float32),
        mesh=mesh,
        scratch_types=[
            pltpu.VMEM((_CHUNK,), jnp.int32),
            pltpu.VMEM((_CHUNK, d), jnp.float32),
            pltpu.SemaphoreType.DMA,
        ],
        compiler_params=pltpu.CompilerParams(use_tc_tiling_on_sc=False),
    )


def kernel(x, embedding):
    b, h = x.shape
    v, d = embedding.shape
    n = b * h
    idx = x.reshape(n).astype(jnp.int32)
    out = _gather_call(n, v, d)(idx, embedding)
    return out.reshape(b, h, d)


# speedup vs baseline: 5.0216x; 1.0445x over previous
import functools

import jax
import jax.numpy as jnp
from jax import lax
from jax.experimental import pallas as pl
from jax.experimental.pallas import tpu as pltpu
from jax.experimental.pallas import tpu_sc as plsc

_NUM_CORES = 2
_NUM_SUBCORES = 16
_NUM_WORKERS = _NUM_CORES * _NUM_SUBCORES

_CHUNK = 1024


@functools.cache
def _gather_call(n: int, v: int, d: int):
    assert n % (_NUM_WORKERS * _CHUNK) == 0
    per_worker = n // _NUM_WORKERS
    n_chunks = per_worker // _CHUNK
    assert n_chunks % 2 == 0 and n_chunks >= 4
    mesh = plsc.VectorSubcoreMesh(core_axis_name="c", subcore_axis_name="s")

    def body(idx_hbm, table_hbm, out_hbm, idx_v, rows_v, sem_l, sem_g, sem_s):
        wid = lax.axis_index("s") * _NUM_CORES + lax.axis_index("c")
        base = wid * per_worker

        def l_copy(i, b):
            return pltpu.make_async_copy(
                idx_hbm.at[pl.ds(base + i * _CHUNK, _CHUNK)],
                idx_v.at[b], sem_l.at[b])

        def g_copy(b):
            return pltpu.make_async_copy(
                table_hbm.at[idx_v.at[b]], rows_v.at[b], sem_g.at[b])

        def s_copy(i, b):
            return pltpu.make_async_copy(
                rows_v.at[b],
                out_hbm.at[pl.ds(base + i * _CHUNK, _CHUNK)], sem_s.at[b])

        l_copy(0, 0).start()
        l_copy(1, 1).start()
        l_copy(0, 0).wait()
        g_copy(0).start()
        g_copy(0).wait()
        s_copy(0, 0).start()
        l_copy(2, 0).start()
        l_copy(1, 1).wait()
        g_copy(1).start()

        def steady(j, carry):
            i0 = 2 * j
            i1 = i0 + 1
            g_copy(1).wait()
            s_copy(i0 - 1, 1).start()
            l_copy(i0 + 1, 1).start()
            l_copy(i0, 0).wait()
            s_copy(i0 - 2, 0).wait()
            g_copy(0).start()
            g_copy(0).wait()
            s_copy(i0, 0).start()
            l_copy(i1 + 1, 0).start()
            l_copy(i1, 1).wait()
            s_copy(i1 - 2, 1).wait()
            g_copy(1).start()
            return carry

        lax.fori_loop(1, n_chunks // 2, steady, 0)

        last = n_chunks - 1
        g_copy(1).wait()
        s_copy(last, 1).start()
        l_copy(n_chunks, 0).wait()
        s_copy(last - 1, 0).wait()
        s_copy(last, 1).wait()

    return pl.kernel(
        body,
        out_type=jax.ShapeDtypeStruct((n, d), jnp.float32),
        mesh=mesh,
        scratch_types=[
            pltpu.VMEM((2, _CHUNK), jnp.int32),
            pltpu.VMEM((2, _CHUNK, d), jnp.float32),
            pltpu.SemaphoreType.DMA((2,)),
            pltpu.SemaphoreType.DMA((2,)),
            pltpu.SemaphoreType.DMA((2,)),
        ],
        compiler_params=pltpu.CompilerParams(use_tc_tiling_on_sc=False),
    )


def kernel(x, embedding):
    b, h = x.shape
    v, d = embedding.shape
    n = b * h
    idx = x.reshape(n).astype(jnp.int32)
    idx = jnp.concatenate([idx, jnp.zeros((_CHUNK,), jnp.int32)])
    out = _gather_call(n, v, d)(idx, embedding)
    return out.reshape(b, h, d)
